# Initial kernel scaffold; baseline (speedup 1.0000x reference)
import jax
import jax.numpy as jnp
from jax.experimental import pallas as pl


def kernel(X_index, weight):
    raise NotImplementedError("write your pallas kernel here")


# speedup vs baseline: 1.6833x; 1.6833x over previous
import functools

import jax
import jax.numpy as jnp
from jax import lax
from jax.experimental import pallas as pl
from jax.experimental.pallas import tpu as pltpu
from jax.experimental.pallas import tpu_sc as plsc

VOCAB = 1000000
EMBED_DIM = 64
BATCH = 16384
HIST = 50

_NC = 2
_NS = 16
_NW = _NC * _NS

_TOTAL = BATCH * HIST
_CL = 128
_CHUNKS = _TOTAL // (_NW * _CL)


def _emb_body(idx_hbm, table_hbm, out_hbm, idx_v, rows_v, gsem):
    c = lax.axis_index("c")
    s = lax.axis_index("s")
    wid = s * _NC + c

    pltpu.sync_copy(idx_hbm.at[wid], idx_v)

    base = wid * _CHUNKS * _CL

    @pl.loop(0, _CHUNKS)
    def _chunk(j):
        pltpu.async_copy(table_hbm.at[idx_v.at[j]], rows_v, gsem).wait()
        pltpu.sync_copy(rows_v, out_hbm.at[pl.ds(base + j * _CL, _CL)])


@jax.jit
def _embed(idx, weight):
    grid_kernel = functools.partial(
        pl.kernel,
        out_type=jax.ShapeDtypeStruct((_TOTAL, EMBED_DIM), jnp.float32),
        mesh=plsc.VectorSubcoreMesh(core_axis_name="c", subcore_axis_name="s"),
        scratch_types=[
            pltpu.VMEM((_CHUNKS, _CL), jnp.int32),
            pltpu.VMEM((_CL, EMBED_DIM), jnp.float32),
            pltpu.SemaphoreType.DMA,
        ],
        compiler_params=pltpu.CompilerParams(use_tc_tiling_on_sc=False),
    )
    out = grid_kernel(_emb_body)(idx, weight)
    return out


def kernel(X_index, weight):
    idx = X_index.reshape(_NW, _CHUNKS, _CL)
    out = _embed(idx, weight)
    return out.reshape(BATCH, HIST, EMBED_DIM)


# speedup vs baseline: 1.8721x; 1.1122x over previous
import functools

import jax
import jax.numpy as jnp
from jax import lax
from jax.experimental import pallas as pl
from jax.experimental.pallas import tpu as pltpu
from jax.experimental.pallas import tpu_sc as plsc

VOCAB = 1000000
EMBED_DIM = 64
BATCH = 16384
HIST = 50

_NC = 2
_NS = 16
_NW = _NC * _NS

_TOTAL = BATCH * HIST
_CL = 128
_NBUF = 4
_CHUNKS = _TOTAL // (_NW * _CL)
_G = _CHUNKS // _NBUF
_GROUP_ROWS = _NBUF * _CL


def _emb_body(idx_hbm, table_hbm, out_hbm,
              idx_v, rows0, rows1, gs0, gs1, ws0, ws1):
    c = lax.axis_index("c")
    s = lax.axis_index("s")
    wid = s * _NC + c

    pltpu.sync_copy(idx_hbm.at[wid], idx_v)

    base = wid * _CHUNKS * _CL
    rows = (rows0, rows1)
    gs = (gs0, gs1)
    ws = (ws0, ws1)

    def fire_group(g, p):
        for b in range(_NBUF):
            pltpu.async_copy(
                table_hbm.at[idx_v.at[g * _NBUF + b]],
                rows[p].at[pl.ds(b * _CL, _CL)],
                gs[p],
            )

    def drain_gathers(p):
        pltpu.make_async_copy(
            table_hbm.at[pl.ds(0, _GROUP_ROWS)], rows[p], gs[p]
        ).wait()

    def fire_write(g, p):
        pltpu.async_copy(
            rows[p], out_hbm.at[pl.ds(base + g * _GROUP_ROWS, _GROUP_ROWS)],
            ws[p],
        )

    def drain_write(p):
        pltpu.make_async_copy(
            rows[p], out_hbm.at[pl.ds(base, _GROUP_ROWS)], ws[p]
        ).wait()

    fire_group(0, 0)

    @pl.loop(0, _G, step=2)
    def _outer(g0):
        @pl.when(g0 > 0)
        def _():
            drain_write(1)
        fire_group(g0 + 1, 1)
        drain_gathers(0)
        fire_write(g0, 0)
        @pl.when(g0 + 2 < _G)
        def _():
            drain_write(0)
            fire_group(g0 + 2, 0)
        drain_gathers(1)
        fire_write(g0 + 1, 1)

    drain_write(0)
    drain_write(1)


@jax.jit
def _embed(idx, weight):
    grid_kernel = functools.partial(
        pl.kernel,
        out_type=jax.ShapeDtypeStruct((_TOTAL, EMBED_DIM), jnp.float32),
        mesh=plsc.VectorSubcoreMesh(core_axis_name="c", subcore_axis_name="s"),
        scratch_types=[
            pltpu.VMEM((_CHUNKS, _CL), jnp.int32),
            pltpu.VMEM((_GROUP_ROWS, EMBED_DIM), jnp.float32),
            pltpu.VMEM((_GROUP_ROWS, EMBED_DIM), jnp.float32),
            pltpu.SemaphoreType.DMA,
            pltpu.SemaphoreType.DMA,
            pltpu.SemaphoreType.DMA,
            pltpu.SemaphoreType.DMA,
        ],
        compiler_params=pltpu.CompilerParams(use_tc_tiling_on_sc=False),
    )
    out = grid_kernel(_emb_body)(idx, weight)
    return out


def kernel(X_index, weight):
    idx = X_index.reshape(_NW, _CHUNKS, _CL)
    out = _embed(idx, weight)
    return out.reshape(BATCH, HIST, EMBED_DIM)
